# baseline (speedup 1.0000x reference)
import jax
import jax.numpy as jnp
from jax.experimental import pallas as pl
from jax.experimental.pallas import tpu as pltpu

B, T, D = 4, 2048, 1024
NS, NR, K = 2, 8, 2
M_BLK = 512
C_TILE = 256
N_TILES = D // C_TILE


def _fused_kernel(x_ref, v_ref, sw_ref, rw_ref, w1_ref, w2_ref, out_ref,
                  w1b_ref):
    @pl.when(pl.program_id(0) == 0)
    def _init():
        w1b_ref[...] = w1_ref[...].astype(jnp.bfloat16)

    x = x_ref[...].astype(jnp.bfloat16)
    w2 = w2_ref[...].astype(jnp.bfloat16)
    logits = jnp.zeros((NR, M_BLK), jnp.float32)
    for c in range(N_TILES):
        sl = slice(c * C_TILE, (c + 1) * C_TILE)
        hc = jax.lax.dot_general(
            x, w1b_ref[sl, :],
            dimension_numbers=(((1,), (1,)), ((), ())),
            preferred_element_type=jnp.float32,
        )
        hc = jnp.maximum(hc, 0.0).astype(jnp.bfloat16)
        logits = logits + jax.lax.dot_general(
            w2[:, sl], hc,
            dimension_numbers=(((1,), (1,)), ((), ())),
            preferred_element_type=jnp.float32,
        )

    m = jnp.max(logits, axis=0, keepdims=True)
    e = jnp.exp(logits - m)
    w = e / jnp.sum(e, axis=0, keepdims=True)

    rows = jax.lax.broadcasted_iota(jnp.int32, w.shape, 0)
    m1 = jnp.max(w, axis=0, keepdims=True)
    c1 = jnp.min(jnp.where(w == m1, rows, NR), axis=0, keepdims=True)
    mask1 = rows == c1
    w_rest = jnp.where(mask1, -jnp.inf, w)
    m2 = jnp.max(w_rest, axis=0, keepdims=True)
    c2 = jnp.min(jnp.where(w_rest == m2, rows, NR), axis=0, keepdims=True)
    sw = jnp.where(mask1 | (rows == c2), w, 0.0)

    comb = jax.lax.dot_general(
        sw, rw_ref[...],
        dimension_numbers=(((0,), (0,)), ((), ())),
        preferred_element_type=jnp.float32,
    )
    wsum = jnp.sum(sw_ref[...], axis=0, keepdims=True)
    v = v_ref[...].reshape(-1, 1)
    out_ref[...] = v * (wsum + comb)


def kernel(gene_embedded, value, shared_W, routing_W, router_W1, router_W2):
    N = B * T
    x = gene_embedded.reshape(N, D)
    v = value.reshape(N)
    grid = N // M_BLK
    out = pl.pallas_call(
        _fused_kernel,
        grid=(grid,),
        in_specs=[
            pl.BlockSpec((M_BLK, D), lambda i: (i, 0)),
            pl.BlockSpec((M_BLK,), lambda i: (i,)),
            pl.BlockSpec((NS, D), lambda i: (0, 0)),
            pl.BlockSpec((NR, D), lambda i: (0, 0)),
            pl.BlockSpec((D, D), lambda i: (0, 0)),
            pl.BlockSpec((NR, D), lambda i: (0, 0)),
        ],
        out_specs=pl.BlockSpec((M_BLK, D), lambda i: (i, 0)),
        out_shape=jax.ShapeDtypeStruct((N, D), jnp.float32),
        scratch_shapes=[pltpu.VMEM((D, D), jnp.bfloat16)],
    )(x, v, shared_W, routing_W, router_W1, router_W2)
    return out.reshape(B, T, D)


# speedup vs baseline: 1.2784x; 1.2784x over previous
import jax
import jax.numpy as jnp
from jax.experimental import pallas as pl
from jax.experimental.pallas import tpu as pltpu

B, T, D = 4, 2048, 1024
NS, NR, K = 2, 8, 2
M_BLK = 512
C_TILE = 256
N_TILES = D // C_TILE


def _fused_kernel(x_ref, v_ref, sw_ref, rw_ref, w1_ref, w2_ref, out_ref,
                  w1b_ref):
    @pl.when(pl.program_id(0) == 0)
    def _init():
        w1b_ref[...] = w1_ref[...].astype(jnp.bfloat16)

    x = x_ref[...].astype(jnp.bfloat16)
    w2 = w2_ref[...].astype(jnp.bfloat16)
    h = jax.lax.dot_general(
        x, w1b_ref[...],
        dimension_numbers=(((1,), (1,)), ((), ())),
        preferred_element_type=jnp.float32,
    )
    h = jnp.maximum(h, 0.0).astype(jnp.bfloat16)
    logits = jax.lax.dot_general(
        w2, h,
        dimension_numbers=(((1,), (1,)), ((), ())),
        preferred_element_type=jnp.float32,
    )

    m = jnp.max(logits, axis=0, keepdims=True)
    e = jnp.exp(logits - m)
    w = e / jnp.sum(e, axis=0, keepdims=True)

    rows = jax.lax.broadcasted_iota(jnp.int32, w.shape, 0)
    m1 = jnp.max(w, axis=0, keepdims=True)
    c1 = jnp.min(jnp.where(w == m1, rows, NR), axis=0, keepdims=True)
    mask1 = rows == c1
    w_rest = jnp.where(mask1, -jnp.inf, w)
    m2 = jnp.max(w_rest, axis=0, keepdims=True)
    c2 = jnp.min(jnp.where(w_rest == m2, rows, NR), axis=0, keepdims=True)
    sw = jnp.where(mask1 | (rows == c2), w, 0.0)

    comb = jax.lax.dot_general(
        sw.astype(jnp.bfloat16), rw_ref[...].astype(jnp.bfloat16),
        dimension_numbers=(((0,), (0,)), ((), ())),
        preferred_element_type=jnp.float32,
    )
    wsum = jnp.sum(sw_ref[...], axis=0, keepdims=True)
    v = v_ref[...].reshape(-1, 1)
    out_ref[...] = v * (wsum + comb)


def kernel(gene_embedded, value, shared_W, routing_W, router_W1, router_W2):
    N = B * T
    x = gene_embedded.reshape(N, D)
    v = value.reshape(N)
    grid = N // M_BLK
    out = pl.pallas_call(
        _fused_kernel,
        grid=(grid,),
        in_specs=[
            pl.BlockSpec((M_BLK, D), lambda i: (i, 0)),
            pl.BlockSpec((M_BLK,), lambda i: (i,)),
            pl.BlockSpec((NS, D), lambda i: (0, 0)),
            pl.BlockSpec((NR, D), lambda i: (0, 0)),
            pl.BlockSpec((D, D), lambda i: (0, 0)),
            pl.BlockSpec((NR, D), lambda i: (0, 0)),
        ],
        out_specs=pl.BlockSpec((M_BLK, D), lambda i: (i, 0)),
        out_shape=jax.ShapeDtypeStruct((N, D), jnp.float32),
        scratch_shapes=[pltpu.VMEM((D, D), jnp.bfloat16)],
    )(x, v, shared_W, routing_W, router_W1, router_W2)
    return out.reshape(B, T, D)


# speedup vs baseline: 1.2795x; 1.0008x over previous
import jax
import jax.numpy as jnp
from jax.experimental import pallas as pl
from jax.experimental.pallas import tpu as pltpu

B, T, D = 4, 2048, 1024
NS, NR, K = 2, 8, 2
M_BLK = 512
GRID = (B * T) // M_BLK


def _fused_kernel(x_ref, v_ref, sw_ref, rw_ref, w1_ref, w2_ref, out_ref,
                  w1b_ref, lg_ref):
    i = pl.program_id(0)
    p = jax.lax.rem(i, 2)

    @pl.when(i == 0)
    def _init():
        w1b_ref[...] = w1_ref[...].astype(jnp.bfloat16)

    @pl.when(i < GRID)
    def _producer():
        x = x_ref[...].astype(jnp.bfloat16)
        h = jax.lax.dot_general(
            x, w1b_ref[...],
            dimension_numbers=(((1,), (1,)), ((), ())),
            preferred_element_type=jnp.float32,
        )
        h = jnp.maximum(h, 0.0).astype(jnp.bfloat16)
        logits = jax.lax.dot_general(
            w2_ref[...].astype(jnp.bfloat16), h,
            dimension_numbers=(((1,), (1,)), ((), ())),
            preferred_element_type=jnp.float32,
        )
        lg_ref[p] = logits

    @pl.when(i > 0)
    def _consumer():
        logits = lg_ref[1 - p]
        m = jnp.max(logits, axis=0, keepdims=True)
        e = jnp.exp(logits - m)
        w = e / jnp.sum(e, axis=0, keepdims=True)

        rows = jax.lax.broadcasted_iota(jnp.int32, w.shape, 0)
        m1 = jnp.max(w, axis=0, keepdims=True)
        c1 = jnp.min(jnp.where(w == m1, rows, NR), axis=0, keepdims=True)
        mask1 = rows == c1
        w_rest = jnp.where(mask1, -jnp.inf, w)
        m2 = jnp.max(w_rest, axis=0, keepdims=True)
        c2 = jnp.min(jnp.where(w_rest == m2, rows, NR), axis=0, keepdims=True)
        sw = jnp.where(mask1 | (rows == c2), w, 0.0)

        comb = jax.lax.dot_general(
            sw.astype(jnp.bfloat16), rw_ref[...].astype(jnp.bfloat16),
            dimension_numbers=(((0,), (0,)), ((), ())),
            preferred_element_type=jnp.float32,
        )
        wsum = jnp.sum(sw_ref[...], axis=0, keepdims=True)
        v = v_ref[...].reshape(-1, 1)
        out_ref[...] = v * (wsum + comb)


def kernel(gene_embedded, value, shared_W, routing_W, router_W1, router_W2):
    N = B * T
    x = gene_embedded.reshape(N, D)
    v = value.reshape(N)
    out = pl.pallas_call(
        _fused_kernel,
        grid=(GRID + 1,),
        in_specs=[
            pl.BlockSpec((M_BLK, D), lambda i: (jnp.minimum(i, GRID - 1), 0)),
            pl.BlockSpec((M_BLK,), lambda i: (jnp.maximum(i - 1, 0),)),
            pl.BlockSpec((NS, D), lambda i: (0, 0)),
            pl.BlockSpec((NR, D), lambda i: (0, 0)),
            pl.BlockSpec((D, D), lambda i: (0, 0)),
            pl.BlockSpec((NR, D), lambda i: (0, 0)),
        ],
        out_specs=pl.BlockSpec((M_BLK, D), lambda i: (jnp.maximum(i - 1, 0), 0)),
        out_shape=jax.ShapeDtypeStruct((N, D), jnp.float32),
        scratch_shapes=[
            pltpu.VMEM((D, D), jnp.bfloat16),
            pltpu.VMEM((2, NR, M_BLK), jnp.float32),
        ],
    )(x, v, shared_W, routing_W, router_W1, router_W2)
    return out.reshape(B, T, D)


# speedup vs baseline: 1.2882x; 1.0068x over previous
import jax
import jax.numpy as jnp
from jax.experimental import pallas as pl
from jax.experimental.pallas import tpu as pltpu

B, T, D = 4, 2048, 1024
NS, NR, K = 2, 8, 2
M_BLK = 512
GRID = (B * T) // M_BLK


def _fused_kernel(x_ref, v_ref, sw_ref, rw_ref, w1_ref, w2_ref, out_ref,
                  w1b_ref, lg_ref):
    i = pl.program_id(0)
    p = jax.lax.rem(i, 2)

    @pl.when(i == 0)
    def _init():
        w1b_ref[...] = w1_ref[...].astype(jnp.bfloat16)

    x = x_ref[...].astype(jnp.bfloat16)
    h = jax.lax.dot_general(
        x, w1b_ref[...],
        dimension_numbers=(((1,), (1,)), ((), ())),
        preferred_element_type=jnp.float32,
    )
    h = jnp.maximum(h, 0.0).astype(jnp.bfloat16)
    new_logits = jax.lax.dot_general(
        w2_ref[...].astype(jnp.bfloat16), h,
        dimension_numbers=(((1,), (1,)), ((), ())),
        preferred_element_type=jnp.float32,
    )

    logits = lg_ref[1 - p]
    m = jnp.max(logits, axis=0, keepdims=True)
    e = jnp.exp(logits - m)
    w = e / jnp.sum(e, axis=0, keepdims=True)

    rows = jax.lax.broadcasted_iota(jnp.int32, w.shape, 0)
    m1 = jnp.max(w, axis=0, keepdims=True)
    c1 = jnp.min(jnp.where(w == m1, rows, NR), axis=0, keepdims=True)
    mask1 = rows == c1
    w_rest = jnp.where(mask1, -jnp.inf, w)
    m2 = jnp.max(w_rest, axis=0, keepdims=True)
    c2 = jnp.min(jnp.where(w_rest == m2, rows, NR), axis=0, keepdims=True)
    sw = jnp.where(mask1 | (rows == c2), w, 0.0)

    comb = jax.lax.dot_general(
        sw.astype(jnp.bfloat16), rw_ref[...].astype(jnp.bfloat16),
        dimension_numbers=(((0,), (0,)), ((), ())),
        preferred_element_type=jnp.float32,
    )
    wsum = jnp.sum(sw_ref[...], axis=0, keepdims=True)
    v = v_ref[...].reshape(-1, 1)
    out_ref[...] = v * (wsum + comb)
    lg_ref[p] = new_logits


def kernel(gene_embedded, value, shared_W, routing_W, router_W1, router_W2):
    N = B * T
    x = gene_embedded.reshape(N, D)
    v = value.reshape(N)
    out = pl.pallas_call(
        _fused_kernel,
        grid=(GRID + 1,),
        in_specs=[
            pl.BlockSpec((M_BLK, D), lambda i: (jnp.minimum(i, GRID - 1), 0)),
            pl.BlockSpec((M_BLK,), lambda i: (jnp.maximum(i - 1, 0),)),
            pl.BlockSpec((NS, D), lambda i: (0, 0)),
            pl.BlockSpec((NR, D), lambda i: (0, 0)),
            pl.BlockSpec((D, D), lambda i: (0, 0)),
            pl.BlockSpec((NR, D), lambda i: (0, 0)),
        ],
        out_specs=pl.BlockSpec((M_BLK, D), lambda i: (jnp.maximum(i - 1, 0), 0)),
        out_shape=jax.ShapeDtypeStruct((N, D), jnp.float32),
        scratch_shapes=[
            pltpu.VMEM((D, D), jnp.bfloat16),
            pltpu.VMEM((2, NR, M_BLK), jnp.float32),
        ],
    )(x, v, shared_W, routing_W, router_W1, router_W2)
    return out.reshape(B, T, D)


# speedup vs baseline: 1.3704x; 1.0638x over previous
import jax
import jax.numpy as jnp
from jax.experimental import pallas as pl
from jax.experimental.pallas import tpu as pltpu

B, T, D = 4, 2048, 1024
NS, NR, K = 2, 8, 2
M_BLK = 1024
GRID = (B * T) // M_BLK


def _fused_kernel(x_ref, v_ref, sw_ref, rw_ref, w1_ref, w2_ref, out_ref,
                  w1b_ref, lg_ref):
    i = pl.program_id(0)
    p = jax.lax.rem(i, 2)

    @pl.when(i == 0)
    def _init():
        w1b_ref[...] = w1_ref[...].astype(jnp.bfloat16)

    x = x_ref[...].astype(jnp.bfloat16)
    h = jax.lax.dot_general(
        x, w1b_ref[...],
        dimension_numbers=(((1,), (1,)), ((), ())),
        preferred_element_type=jnp.float32,
    )
    h = jnp.maximum(h, 0.0).astype(jnp.bfloat16)
    new_logits = jax.lax.dot_general(
        w2_ref[...].astype(jnp.bfloat16), h,
        dimension_numbers=(((1,), (1,)), ((), ())),
        preferred_element_type=jnp.float32,
    )

    logits = lg_ref[1 - p]
    m = jnp.max(logits, axis=0, keepdims=True)
    e = jnp.exp(logits - m)
    w = e / jnp.sum(e, axis=0, keepdims=True)

    rows = jax.lax.broadcasted_iota(jnp.int32, w.shape, 0)
    m1 = jnp.max(w, axis=0, keepdims=True)
    c1 = jnp.min(jnp.where(w == m1, rows, NR), axis=0, keepdims=True)
    mask1 = rows == c1
    w_rest = jnp.where(mask1, -jnp.inf, w)
    m2 = jnp.max(w_rest, axis=0, keepdims=True)
    c2 = jnp.min(jnp.where(w_rest == m2, rows, NR), axis=0, keepdims=True)
    sw = jnp.where(mask1 | (rows == c2), w, 0.0)

    comb = jax.lax.dot_general(
        sw.astype(jnp.bfloat16), rw_ref[...].astype(jnp.bfloat16),
        dimension_numbers=(((0,), (0,)), ((), ())),
        preferred_element_type=jnp.float32,
    )
    wsum = jnp.sum(sw_ref[...], axis=0, keepdims=True)
    v = v_ref[...].reshape(-1, 1)
    out_ref[...] = v * (wsum + comb)
    lg_ref[p] = new_logits


def kernel(gene_embedded, value, shared_W, routing_W, router_W1, router_W2):
    N = B * T
    x = gene_embedded.reshape(N, D)
    v = value.reshape(N)
    out = pl.pallas_call(
        _fused_kernel,
        grid=(GRID + 1,),
        in_specs=[
            pl.BlockSpec((M_BLK, D), lambda i: (jnp.minimum(i, GRID - 1), 0)),
            pl.BlockSpec((M_BLK,), lambda i: (jnp.maximum(i - 1, 0),)),
            pl.BlockSpec((NS, D), lambda i: (0, 0)),
            pl.BlockSpec((NR, D), lambda i: (0, 0)),
            pl.BlockSpec((D, D), lambda i: (0, 0)),
            pl.BlockSpec((NR, D), lambda i: (0, 0)),
        ],
        out_specs=pl.BlockSpec((M_BLK, D), lambda i: (jnp.maximum(i - 1, 0), 0)),
        out_shape=jax.ShapeDtypeStruct((N, D), jnp.float32),
        scratch_shapes=[
            pltpu.VMEM((D, D), jnp.bfloat16),
            pltpu.VMEM((2, NR, M_BLK), jnp.float32),
        ],
    )(x, v, shared_W, routing_W, router_W1, router_W2)
    return out.reshape(B, T, D)
